# baseline (speedup 1.0000x reference)
import functools
import math

import jax
import jax.numpy as jnp
from jax import lax
from jax.experimental import pallas as pl
from jax.experimental.pallas import tpu as pltpu
from jax.experimental.pallas import tpu_sc as plsc

N = 10000
E = 320000
D = 128
D_ATTR = 16
D_EMB = 16
FC_HIDDEN = 8

NC = 2
NS = 16
NW = NC * NS
PER_W = E // NW
CHUNK = 80
ITERS = PER_W // CHUNK
NBUF = 5
DW = D // 2
CH_S = 40
IT_S = PER_W // CH_S
NBUF_S = 2

INV_SQRT_D = 1.0 / math.sqrt(D)
INV_SQRT_EMB = 1.0 / math.sqrt(D_EMB)
INV_SQRT_FC = 1.0 / math.sqrt(FC_HIDDEN)
INV_SQRT_AVG = 1.0 / math.sqrt(32.0)
INV_SQRT_ATTR = 1.0 / math.sqrt(D_ATTR)
INV_SQRT_DDA = 1.0 / math.sqrt(D * D_ATTR)


def _silu(v):
    return v * jax.nn.sigmoid(v)


def _x_body(nf_ref, w1_ref, o_ref):
    o_ref[...] = (jnp.dot(nf_ref[...], w1_ref[...],
                          preferred_element_type=jnp.float32)
                  * INV_SQRT_D).astype(jnp.bfloat16)


def _compute_x(node_feats, W1):
    BN = 1000
    return pl.pallas_call(
        _x_body,
        grid=(N // BN,),
        in_specs=[pl.BlockSpec((BN, D), lambda i: (i, 0)),
                  pl.BlockSpec((D, D), lambda i: (0, 0))],
        out_specs=pl.BlockSpec((BN, D), lambda i: (i, 0)),
        out_shape=jax.ShapeDtypeStruct((N, D), jnp.bfloat16),
    )(node_feats, W1)


@functools.lru_cache(maxsize=None)
def _sc_mesh():
    return plsc.VectorSubcoreMesh(core_axis_name="c", subcore_axis_name="s",
                                  num_cores=NC, num_subcores=NS)


@functools.lru_cache(maxsize=None)
def _make_gather():
    @functools.partial(
        pl.kernel,
        out_type=jax.ShapeDtypeStruct((E, DW), jnp.int32),
        mesh=_sc_mesh(),
        scratch_types=[
            pltpu.VMEM((ITERS, CHUNK), jnp.int32),
            pltpu.VMEM((NBUF, CHUNK, DW), jnp.int32),
        ] + [pltpu.SemaphoreType.DMA] * NBUF,
        compiler_params=pltpu.CompilerParams(use_tc_tiling_on_sc=False),
    )
    def _gather_k(table_hbm, idx3_hbm, out_hbm, idx_all, rows, *sems):
        wid = lax.axis_index("s") * NC + lax.axis_index("c")
        base = pl.multiple_of(wid * PER_W, CHUNK)

        pltpu.sync_copy(idx3_hbm.at[wid], idx_all)

        def start(j, b):
            pltpu.async_copy(table_hbm.at[idx_all.at[j]], rows.at[b], sems[b])

        def drain(j, b):
            pltpu.make_async_copy(table_hbm.at[idx_all.at[j]], rows.at[b],
                                  sems[b]).wait()
            off = pl.multiple_of(base + j * CHUNK, CHUNK)
            pltpu.sync_copy(rows.at[b], out_hbm.at[pl.ds(off, CHUNK), :])

        for b in range(NBUF):
            start(b, b)

        def body(g, carry):
            for b in range(NBUF):
                j = NBUF * g + b
                drain(j, b)
                start(j + NBUF, b)
            return carry

        lax.fori_loop(0, ITERS // NBUF - 1, body, 0)
        for b in range(NBUF):
            drain(ITERS - NBUF + b, b)

    return _gather_k


def _edge_body(emb_ref, ea_ref, xg_ref, m1_ref, m2_ref, w2_ref, o_ref):
    h = _silu(jnp.dot(emb_ref[...], m1_ref[...],
                      preferred_element_type=jnp.float32) * INV_SQRT_EMB)
    w = jnp.dot(h, m2_ref[...],
                preferred_element_type=jnp.float32) * INV_SQRT_FC
    m = xg_ref[...].astype(jnp.float32) * ea_ref[...] * w
    m = jnp.dot(m.astype(jnp.bfloat16), w2_ref[...],
                preferred_element_type=jnp.float32) * INV_SQRT_D
    o_ref[...] = _silu(m)


def _compute_msg(edge_embedding, edge_attrs, xg, M1, M2, W2b):
    BE = 2000
    return pl.pallas_call(
        _edge_body,
        grid=(E // BE,),
        in_specs=[pl.BlockSpec((BE, D_EMB), lambda i: (i, 0)),
                  pl.BlockSpec((BE, 1), lambda i: (i, 0)),
                  pl.BlockSpec((BE, D), lambda i: (i, 0)),
                  pl.BlockSpec((D_EMB, FC_HIDDEN), lambda i: (0, 0)),
                  pl.BlockSpec((FC_HIDDEN, D), lambda i: (0, 0)),
                  pl.BlockSpec((D, D), lambda i: (0, 0))],
        out_specs=pl.BlockSpec((BE, D), lambda i: (i, 0)),
        out_shape=jax.ShapeDtypeStruct((E, D), jnp.float32),
    )(edge_embedding, edge_attrs, xg, M1, M2, W2b)


N_PAD = 10240
ROWS_PER_SUB = N_PAD // NS


@functools.lru_cache(maxsize=None)
def _make_scatter():
    @functools.partial(
        pl.kernel,
        out_type=jax.ShapeDtypeStruct((NC, N_PAD, D), jnp.float32),
        mesh=_sc_mesh(),
        scratch_types=[
            pltpu.VMEM((IT_S, CH_S), jnp.int32),
            pltpu.VMEM((NBUF_S, CH_S, D), jnp.float32),
            pltpu.VMEM_SHARED((N_PAD, D), jnp.float32),
        ] + [pltpu.SemaphoreType.DMA] * NBUF_S,
    )
    def _scatter_k(msg_hbm, dst3_hbm, zeros_hbm, out_hbm, idx_all, msgb, acc,
                   *sems):
        cid = lax.axis_index("c")
        sid = lax.axis_index("s")
        wid = sid * NC + cid
        base = pl.multiple_of(wid * PER_W, CH_S)

        pltpu.sync_copy(dst3_hbm.at[wid], idx_all)

        zbase = pl.multiple_of(sid * ROWS_PER_SUB, 8)
        pltpu.sync_copy(zeros_hbm.at[pl.ds(zbase, ROWS_PER_SUB), :],
                        acc.at[pl.ds(zbase, ROWS_PER_SUB), :])
        plsc.subcore_barrier()

        def start(j, b):
            off = pl.multiple_of(base + j * CH_S, CH_S)
            pltpu.async_copy(msg_hbm.at[pl.ds(off, CH_S), :], msgb.at[b],
                             sems[b])

        def drain(j, b):
            off = pl.multiple_of(base + j * CH_S, CH_S)
            pltpu.make_async_copy(msg_hbm.at[pl.ds(off, CH_S), :],
                                  msgb.at[b], sems[b]).wait()
            pltpu.sync_copy(msgb.at[b], acc.at[idx_all.at[j]], add=True)

        for b in range(NBUF_S):
            start(b, b)

        def body(g, carry):
            for b in range(NBUF_S):
                j = NBUF_S * g + b
                drain(j, b)
                start(j + NBUF_S, b)
            return carry

        lax.fori_loop(0, IT_S // NBUF_S - 1, body, 0)
        for b in range(NBUF_S):
            drain(IT_S - NBUF_S + b, b)
        plsc.subcore_barrier()

        pltpu.sync_copy(acc.at[pl.ds(zbase, ROWS_PER_SUB), :],
                        out_hbm.at[cid, pl.ds(zbase, ROWS_PER_SUB), :])

    return _scatter_k


def _upd_body(acc_ref, na_ref, nf_ref, wtpt_ref, w3_ref, w2d_ref, o_ref):
    na = na_ref[...]
    agg = (acc_ref[0] + acc_ref[1]) * INV_SQRT_AVG
    t = jnp.dot(na, wtpt_ref[...], preferred_element_type=jnp.float32)
    upd = agg * t * INV_SQRT_ATTR
    upd = jnp.dot(upd.astype(jnp.bfloat16), w3_ref[...],
                  preferred_element_type=jnp.float32) * INV_SQRT_D
    upd = _silu(upd)
    y = jnp.dot(nf_ref[...].astype(jnp.bfloat16), w2d_ref[...],
                preferred_element_type=jnp.float32)
    sc = na[:, 0:1] * y[:, 0:D]
    for j in range(1, D_ATTR):
        sc = sc + na[:, j:j + 1] * y[:, j * D:(j + 1) * D]
    o_ref[...] = upd + sc * INV_SQRT_DDA


def _compute_out(acc2, node_attrs, node_feats, WtpT, W3, W2d):
    BN = 1000
    return pl.pallas_call(
        _upd_body,
        grid=(N // BN,),
        in_specs=[pl.BlockSpec((NC, BN, D), lambda i: (0, i, 0)),
                  pl.BlockSpec((BN, D_ATTR), lambda i: (i, 0)),
                  pl.BlockSpec((BN, D), lambda i: (i, 0)),
                  pl.BlockSpec((D_ATTR, D), lambda i: (0, 0)),
                  pl.BlockSpec((D, D), lambda i: (0, 0)),
                  pl.BlockSpec((D, D_ATTR * D), lambda i: (0, 0))],
        out_specs=pl.BlockSpec((BN, D), lambda i: (i, 0)),
        out_shape=jax.ShapeDtypeStruct((N, D), jnp.float32),
    )(acc2, node_attrs, node_feats, WtpT, W3, W2d)


def kernel(node_feats, node_attrs, edge_embedding, edge_attrs, edge_index,
           W1, M1, M2, W2, Wtp, W3, Wsc):
    src = edge_index[0]
    dst = edge_index[1]

    src3 = src.reshape(NW, ITERS, CHUNK)
    dst3 = dst.reshape(NW, IT_S, CH_S)

    x = _compute_x(node_feats, W1)
    xi = jax.lax.bitcast_convert_type(x.reshape(N, DW, 2),
                                      jnp.int32)
    xgi = _make_gather()(xi, src3)
    xg = jax.lax.bitcast_convert_type(xgi, jnp.bfloat16).reshape(E, D)
    msg = _compute_msg(edge_embedding, edge_attrs, xg, M1, M2,
                       W2.astype(jnp.bfloat16))
    zeros = jnp.zeros((N_PAD, D), jnp.float32)
    acc2 = _make_scatter()(msg, dst3, zeros)

    WtpT = Wtp.T
    W2d = Wsc.transpose(1, 2, 0).reshape(D, D_ATTR * D).astype(jnp.bfloat16)
    return _compute_out(acc2, node_attrs, node_feats, WtpT,
                        W3.astype(jnp.bfloat16), W2d)


# speedup vs baseline: 2.0922x; 2.0922x over previous
import functools
import math

import jax
import jax.numpy as jnp
from jax import lax
from jax.experimental import pallas as pl
from jax.experimental.pallas import tpu as pltpu
from jax.experimental.pallas import tpu_sc as plsc

N = 10000
E = 320000
D = 128
D_ATTR = 16
D_EMB = 16
FC_HIDDEN = 8

NC = 2
NS = 16
NW = NC * NS
PER_W = E // NW
CHUNK = 80
ITERS = PER_W // CHUNK
NBUF = 5
DW = D // 2
CH_S = 40
IT_S = PER_W // CH_S
NBUF_S = 2

INV_SQRT_D = 1.0 / math.sqrt(D)
INV_SQRT_EMB = 1.0 / math.sqrt(D_EMB)
INV_SQRT_FC = 1.0 / math.sqrt(FC_HIDDEN)
INV_SQRT_AVG = 1.0 / math.sqrt(32.0)
INV_SQRT_ATTR = 1.0 / math.sqrt(D_ATTR)
INV_SQRT_DDA = 1.0 / math.sqrt(D * D_ATTR)


def _silu(v):
    return v * jax.nn.sigmoid(v)


def _x_body(nf_ref, w1_ref, o_ref):
    o_ref[...] = jnp.dot(nf_ref[...], w1_ref[...],
                         preferred_element_type=jnp.float32) * INV_SQRT_D


def _compute_x(node_feats, W1):
    BN = 1000
    return pl.pallas_call(
        _x_body,
        grid=(N // BN,),
        in_specs=[pl.BlockSpec((BN, D), lambda i: (i, 0)),
                  pl.BlockSpec((D, D), lambda i: (0, 0))],
        out_specs=pl.BlockSpec((BN, D), lambda i: (i, 0)),
        out_shape=jax.ShapeDtypeStruct((N, D), jnp.float32),
    )(node_feats, W1)


@functools.lru_cache(maxsize=None)
def _sc_mesh():
    return plsc.VectorSubcoreMesh(core_axis_name="c", subcore_axis_name="s",
                                  num_cores=NC, num_subcores=NS)


@functools.lru_cache(maxsize=None)
def _make_gather():
    @functools.partial(
        pl.kernel,
        out_type=jax.ShapeDtypeStruct((E, D), jnp.float32),
        mesh=_sc_mesh(),
        scratch_types=[
            pltpu.VMEM((ITERS, CHUNK), jnp.int32),
            pltpu.VMEM((NBUF, CHUNK, D), jnp.float32),
        ] + [pltpu.SemaphoreType.DMA] * NBUF,
    )
    def _gather_k(table_hbm, idx3_hbm, out_hbm, idx_all, rows, *sems):
        wid = lax.axis_index("s") * NC + lax.axis_index("c")
        base = pl.multiple_of(wid * PER_W, CHUNK)

        pltpu.sync_copy(idx3_hbm.at[wid], idx_all)

        def start(j, b):
            pltpu.async_copy(table_hbm.at[idx_all.at[j]], rows.at[b], sems[b])

        def drain(j, b):
            pltpu.make_async_copy(table_hbm.at[idx_all.at[j]], rows.at[b],
                                  sems[b]).wait()
            off = pl.multiple_of(base + j * CHUNK, CHUNK)
            pltpu.sync_copy(rows.at[b], out_hbm.at[pl.ds(off, CHUNK), :])

        for b in range(NBUF):
            start(b, b)

        def body(g, carry):
            for b in range(NBUF):
                j = NBUF * g + b
                drain(j, b)
                start(j + NBUF, b)
            return carry

        lax.fori_loop(0, ITERS // NBUF - 1, body, 0)
        for b in range(NBUF):
            drain(ITERS - NBUF + b, b)

    return _gather_k


def _edge_body(emb_ref, ea_ref, xg_ref, m1_ref, m2_ref, w2_ref, o_ref):
    h = _silu(jnp.dot(emb_ref[...], m1_ref[...],
                      preferred_element_type=jnp.float32) * INV_SQRT_EMB)
    w = jnp.dot(h, m2_ref[...],
                preferred_element_type=jnp.float32) * INV_SQRT_FC
    m = xg_ref[...] * ea_ref[...] * w
    m = jnp.dot(m.astype(jnp.bfloat16), w2_ref[...],
                preferred_element_type=jnp.float32) * INV_SQRT_D
    o_ref[...] = _silu(m)


def _compute_msg(edge_embedding, edge_attrs, xg, M1, M2, W2b):
    BE = 2000
    return pl.pallas_call(
        _edge_body,
        grid=(E // BE,),
        in_specs=[pl.BlockSpec((BE, D_EMB), lambda i: (i, 0)),
                  pl.BlockSpec((BE, 1), lambda i: (i, 0)),
                  pl.BlockSpec((BE, D), lambda i: (i, 0)),
                  pl.BlockSpec((D_EMB, FC_HIDDEN), lambda i: (0, 0)),
                  pl.BlockSpec((FC_HIDDEN, D), lambda i: (0, 0)),
                  pl.BlockSpec((D, D), lambda i: (0, 0))],
        out_specs=pl.BlockSpec((BE, D), lambda i: (i, 0)),
        out_shape=jax.ShapeDtypeStruct((E, D), jnp.float32),
    )(edge_embedding, edge_attrs, xg, M1, M2, W2b)


N_PAD = 10240
ROWS_PER_SUB = N_PAD // NS


@functools.lru_cache(maxsize=None)
def _make_scatter():
    @functools.partial(
        pl.kernel,
        out_type=jax.ShapeDtypeStruct((NC, N_PAD, D), jnp.float32),
        mesh=_sc_mesh(),
        scratch_types=[
            pltpu.VMEM((IT_S, CH_S), jnp.int32),
            pltpu.VMEM((NBUF_S, CH_S, D), jnp.float32),
            pltpu.VMEM_SHARED((N_PAD, D), jnp.float32),
        ] + [pltpu.SemaphoreType.DMA] * NBUF_S,
    )
    def _scatter_k(msg_hbm, dst3_hbm, zeros_hbm, out_hbm, idx_all, msgb, acc,
                   *sems):
        cid = lax.axis_index("c")
        sid = lax.axis_index("s")
        wid = sid * NC + cid
        base = pl.multiple_of(wid * PER_W, CH_S)

        pltpu.sync_copy(dst3_hbm.at[wid], idx_all)

        zbase = pl.multiple_of(sid * ROWS_PER_SUB, 8)
        pltpu.sync_copy(zeros_hbm.at[pl.ds(zbase, ROWS_PER_SUB), :],
                        acc.at[pl.ds(zbase, ROWS_PER_SUB), :])
        plsc.subcore_barrier()

        def start(j, b):
            off = pl.multiple_of(base + j * CH_S, CH_S)
            pltpu.async_copy(msg_hbm.at[pl.ds(off, CH_S), :], msgb.at[b],
                             sems[b])

        def drain(j, b):
            off = pl.multiple_of(base + j * CH_S, CH_S)
            pltpu.make_async_copy(msg_hbm.at[pl.ds(off, CH_S), :],
                                  msgb.at[b], sems[b]).wait()
            pltpu.sync_copy(msgb.at[b], acc.at[idx_all.at[j]], add=True)

        for b in range(NBUF_S):
            start(b, b)

        def body(g, carry):
            for b in range(NBUF_S):
                j = NBUF_S * g + b
                drain(j, b)
                start(j + NBUF_S, b)
            return carry

        lax.fori_loop(0, IT_S // NBUF_S - 1, body, 0)
        for b in range(NBUF_S):
            drain(IT_S - NBUF_S + b, b)
        plsc.subcore_barrier()

        pltpu.sync_copy(acc.at[pl.ds(zbase, ROWS_PER_SUB), :],
                        out_hbm.at[cid, pl.ds(zbase, ROWS_PER_SUB), :])

    return _scatter_k


def _upd_body(acc_ref, na_ref, nf_ref, wtpt_ref, w3_ref, w2d_ref, o_ref):
    na = na_ref[...]
    agg = (acc_ref[0] + acc_ref[1]) * INV_SQRT_AVG
    t = jnp.dot(na, wtpt_ref[...], preferred_element_type=jnp.float32)
    upd = agg * t * INV_SQRT_ATTR
    upd = jnp.dot(upd.astype(jnp.bfloat16), w3_ref[...],
                  preferred_element_type=jnp.float32) * INV_SQRT_D
    upd = _silu(upd)
    y = jnp.dot(nf_ref[...].astype(jnp.bfloat16), w2d_ref[...],
                preferred_element_type=jnp.float32)
    sc = na[:, 0:1] * y[:, 0:D]
    for j in range(1, D_ATTR):
        sc = sc + na[:, j:j + 1] * y[:, j * D:(j + 1) * D]
    o_ref[...] = upd + sc * INV_SQRT_DDA


def _compute_out(acc2, node_attrs, node_feats, WtpT, W3, W2d):
    BN = 1000
    return pl.pallas_call(
        _upd_body,
        grid=(N // BN,),
        in_specs=[pl.BlockSpec((NC, BN, D), lambda i: (0, i, 0)),
                  pl.BlockSpec((BN, D_ATTR), lambda i: (i, 0)),
                  pl.BlockSpec((BN, D), lambda i: (i, 0)),
                  pl.BlockSpec((D_ATTR, D), lambda i: (0, 0)),
                  pl.BlockSpec((D, D), lambda i: (0, 0)),
                  pl.BlockSpec((D, D_ATTR * D), lambda i: (0, 0))],
        out_specs=pl.BlockSpec((BN, D), lambda i: (i, 0)),
        out_shape=jax.ShapeDtypeStruct((N, D), jnp.float32),
    )(acc2, node_attrs, node_feats, WtpT, W3, W2d)


def kernel(node_feats, node_attrs, edge_embedding, edge_attrs, edge_index,
           W1, M1, M2, W2, Wtp, W3, Wsc):
    src = edge_index[0]
    dst = edge_index[1]

    src3 = src.reshape(NW, ITERS, CHUNK)
    dst3 = dst.reshape(NW, IT_S, CH_S)

    x = _compute_x(node_feats, W1)
    xg = _make_gather()(x, src3)
    msg = _compute_msg(edge_embedding, edge_attrs, xg, M1, M2,
                       W2.astype(jnp.bfloat16))
    zeros = jnp.zeros((N_PAD, D), jnp.float32)
    acc2 = _make_scatter()(msg, dst3, zeros)

    WtpT = Wtp.T
    W2d = Wsc.transpose(1, 2, 0).reshape(D, D_ATTR * D).astype(jnp.bfloat16)
    return _compute_out(acc2, node_attrs, node_feats, WtpT,
                        W3.astype(jnp.bfloat16), W2d)


# speedup vs baseline: 2.1976x; 1.0504x over previous
import functools
import math

import jax
import jax.numpy as jnp
from jax import lax
from jax.experimental import pallas as pl
from jax.experimental.pallas import tpu as pltpu
from jax.experimental.pallas import tpu_sc as plsc

N = 10000
E = 320000
D = 128
D_ATTR = 16
D_EMB = 16
FC_HIDDEN = 8

NC = 2
NS = 16
NW = NC * NS
S = 2
ES = E // S
PER_W = ES // NW
CHUNK = 40
ITERS = PER_W // CHUNK
NBUF = 5

INV_SQRT_D = 1.0 / math.sqrt(D)
INV_SQRT_EMB = 1.0 / math.sqrt(D_EMB)
INV_SQRT_FC = 1.0 / math.sqrt(FC_HIDDEN)
INV_SQRT_AVG = 1.0 / math.sqrt(32.0)
INV_SQRT_ATTR = 1.0 / math.sqrt(D_ATTR)
INV_SQRT_DDA = 1.0 / math.sqrt(D * D_ATTR)


def _silu(v):
    return v * jax.nn.sigmoid(v)


def _x_body(nf_ref, w1_ref, o_ref):
    o_ref[...] = jnp.dot(nf_ref[...], w1_ref[...],
                         preferred_element_type=jnp.float32) * INV_SQRT_D


def _compute_x(node_feats, W1):
    BN = 1000
    return pl.pallas_call(
        _x_body,
        grid=(N // BN,),
        in_specs=[pl.BlockSpec((BN, D), lambda i: (i, 0)),
                  pl.BlockSpec((D, D), lambda i: (0, 0))],
        out_specs=pl.BlockSpec((BN, D), lambda i: (i, 0)),
        out_shape=jax.ShapeDtypeStruct((N, D), jnp.float32),
    )(node_feats, W1)


@functools.lru_cache(maxsize=None)
def _sc_mesh():
    return plsc.VectorSubcoreMesh(core_axis_name="c", subcore_axis_name="s",
                                  num_cores=NC, num_subcores=NS)


@functools.lru_cache(maxsize=None)
def _make_gather():
    @functools.partial(
        pl.kernel,
        out_type=jax.ShapeDtypeStruct((ES, D), jnp.float32),
        mesh=_sc_mesh(),
        scratch_types=[
            pltpu.VMEM((ITERS, CHUNK), jnp.int32),
            pltpu.VMEM((NBUF, CHUNK, D), jnp.float32),
        ] + [pltpu.SemaphoreType.DMA] * NBUF,
    )
    def _gather_k(table_hbm, idx3_hbm, out_hbm, idx_all, rows, *sems):
        wid = lax.axis_index("s") * NC + lax.axis_index("c")
        base = pl.multiple_of(wid * PER_W, CHUNK)

        pltpu.sync_copy(idx3_hbm.at[wid], idx_all)

        def start(j, b):
            pltpu.async_copy(table_hbm.at[idx_all.at[j]], rows.at[b], sems[b])

        def drain(j, b):
            pltpu.make_async_copy(table_hbm.at[idx_all.at[j]], rows.at[b],
                                  sems[b]).wait()
            off = pl.multiple_of(base + j * CHUNK, CHUNK)
            pltpu.sync_copy(rows.at[b], out_hbm.at[pl.ds(off, CHUNK), :])

        for b in range(NBUF):
            start(b, b)

        def body(g, carry):
            for b in range(NBUF):
                j = NBUF * g + b
                drain(j, b)
                start(j + NBUF, b)
            return carry

        lax.fori_loop(0, ITERS // NBUF - 1, body, 0)
        for b in range(NBUF):
            drain(ITERS - NBUF + b, b)

    return _gather_k


def _edge_body(emb_ref, ea_ref, xg_ref, m1_ref, m2_ref, w2_ref, o_ref):
    h = _silu(jnp.dot(emb_ref[...], m1_ref[...],
                      preferred_element_type=jnp.float32) * INV_SQRT_EMB)
    w = jnp.dot(h, m2_ref[...],
                preferred_element_type=jnp.float32) * INV_SQRT_FC
    m = xg_ref[...] * ea_ref[...] * w
    m = jnp.dot(m.astype(jnp.bfloat16), w2_ref[...],
                preferred_element_type=jnp.float32) * INV_SQRT_D
    o_ref[...] = _silu(m)


def _compute_msg(edge_embedding, edge_attrs, xg, M1, M2, W2b):
    BE = 2000
    return pl.pallas_call(
        _edge_body,
        grid=(ES // BE,),
        in_specs=[pl.BlockSpec((BE, D_EMB), lambda i: (i, 0)),
                  pl.BlockSpec((BE, 1), lambda i: (i, 0)),
                  pl.BlockSpec((BE, D), lambda i: (i, 0)),
                  pl.BlockSpec((D_EMB, FC_HIDDEN), lambda i: (0, 0)),
                  pl.BlockSpec((FC_HIDDEN, D), lambda i: (0, 0)),
                  pl.BlockSpec((D, D), lambda i: (0, 0))],
        out_specs=pl.BlockSpec((BE, D), lambda i: (i, 0)),
        out_shape=jax.ShapeDtypeStruct((ES, D), jnp.float32),
    )(edge_embedding, edge_attrs, xg, M1, M2, W2b)


N_PAD = 10240
ROWS_PER_SUB = N_PAD // NS


@functools.lru_cache(maxsize=None)
def _make_scatter():
    @functools.partial(
        pl.kernel,
        out_type=jax.ShapeDtypeStruct((NC, N_PAD, D), jnp.float32),
        mesh=_sc_mesh(),
        scratch_types=[
            pltpu.VMEM((ITERS, CHUNK), jnp.int32),
            pltpu.VMEM((NBUF, CHUNK, D), jnp.float32),
            pltpu.VMEM_SHARED((N_PAD, D), jnp.float32),
        ] + [pltpu.SemaphoreType.DMA] * NBUF,
    )
    def _scatter_k(msg_hbm, dst3_hbm, zeros_hbm, out_hbm, idx_all, msgb, acc,
                   *sems):
        cid = lax.axis_index("c")
        sid = lax.axis_index("s")
        wid = sid * NC + cid
        base = pl.multiple_of(wid * PER_W, CHUNK)

        pltpu.sync_copy(dst3_hbm.at[wid], idx_all)

        zbase = pl.multiple_of(sid * ROWS_PER_SUB, 8)
        pltpu.sync_copy(zeros_hbm.at[pl.ds(zbase, ROWS_PER_SUB), :],
                        acc.at[pl.ds(zbase, ROWS_PER_SUB), :])
        plsc.subcore_barrier()

        def start(j, b):
            off = pl.multiple_of(base + j * CHUNK, CHUNK)
            pltpu.async_copy(msg_hbm.at[pl.ds(off, CHUNK), :], msgb.at[b],
                             sems[b])

        def drain(j, b):
            off = pl.multiple_of(base + j * CHUNK, CHUNK)
            pltpu.make_async_copy(msg_hbm.at[pl.ds(off, CHUNK), :],
                                  msgb.at[b], sems[b]).wait()
            pltpu.sync_copy(msgb.at[b], acc.at[idx_all.at[j]], add=True)

        for b in range(NBUF):
            start(b, b)

        def body(g, carry):
            for b in range(NBUF):
                j = NBUF * g + b
                drain(j, b)
                start(j + NBUF, b)
            return carry

        lax.fori_loop(0, ITERS // NBUF - 1, body, 0)
        for b in range(NBUF):
            drain(ITERS - NBUF + b, b)
        plsc.subcore_barrier()

        pltpu.sync_copy(acc.at[pl.ds(zbase, ROWS_PER_SUB), :],
                        out_hbm.at[cid, pl.ds(zbase, ROWS_PER_SUB), :])

    return _scatter_k


def _upd_body(acc_ref, na_ref, nf_ref, wtpt_ref, w3_ref, w2d_ref, o_ref):
    na = na_ref[...]
    agg = acc_ref[0]
    for k in range(1, S * NC):
        agg = agg + acc_ref[k]
    agg = agg * INV_SQRT_AVG
    t = jnp.dot(na, wtpt_ref[...], preferred_element_type=jnp.float32)
    upd = agg * t * INV_SQRT_ATTR
    upd = jnp.dot(upd.astype(jnp.bfloat16), w3_ref[...],
                  preferred_element_type=jnp.float32) * INV_SQRT_D
    upd = _silu(upd)
    y = jnp.dot(nf_ref[...].astype(jnp.bfloat16), w2d_ref[...],
                preferred_element_type=jnp.float32)
    sc = na[:, 0:1] * y[:, 0:D]
    for j in range(1, D_ATTR):
        sc = sc + na[:, j:j + 1] * y[:, j * D:(j + 1) * D]
    o_ref[...] = upd + sc * INV_SQRT_DDA


def _compute_out(acc2, node_attrs, node_feats, WtpT, W3, W2d):
    BN = 1000
    return pl.pallas_call(
        _upd_body,
        grid=(N // BN,),
        in_specs=[pl.BlockSpec((S * NC, BN, D), lambda i: (0, i, 0)),
                  pl.BlockSpec((BN, D_ATTR), lambda i: (i, 0)),
                  pl.BlockSpec((BN, D), lambda i: (i, 0)),
                  pl.BlockSpec((D_ATTR, D), lambda i: (0, 0)),
                  pl.BlockSpec((D, D), lambda i: (0, 0)),
                  pl.BlockSpec((D, D_ATTR * D), lambda i: (0, 0))],
        out_specs=pl.BlockSpec((BN, D), lambda i: (i, 0)),
        out_shape=jax.ShapeDtypeStruct((N, D), jnp.float32),
    )(acc2, node_attrs, node_feats, WtpT, W3, W2d)


def kernel(node_feats, node_attrs, edge_embedding, edge_attrs, edge_index,
           W1, M1, M2, W2, Wtp, W3, Wsc):
    src = edge_index[0]
    dst = edge_index[1]

    x = _compute_x(node_feats, W1)
    zeros = jnp.zeros((N_PAD, D), jnp.float32)
    W2b = W2.astype(jnp.bfloat16)
    accs = []
    for t in range(S):
        sl = slice(t * ES, (t + 1) * ES)
        src3_t = src[sl].reshape(NW, ITERS, CHUNK)
        dst3_t = dst[sl].reshape(NW, ITERS, CHUNK)
        xg_t = _make_gather()(x, src3_t)
        msg_t = _compute_msg(edge_embedding[sl], edge_attrs[sl], xg_t,
                             M1, M2, W2b)
        accs.append(_make_scatter()(msg_t, dst3_t, zeros))
    acc2 = jnp.concatenate(accs, axis=0)

    WtpT = Wtp.T
    W2d = Wsc.transpose(1, 2, 0).reshape(D, D_ATTR * D).astype(jnp.bfloat16)
    return _compute_out(acc2, node_attrs, node_feats, WtpT,
                        W3.astype(jnp.bfloat16), W2d)
